# baseline (speedup 1.0000x reference)
import functools

import jax
import jax.numpy as jnp
import numpy as np
from jax import lax
from jax.experimental import pallas as pl
from jax.experimental.pallas import tpu as pltpu
from jax.experimental.pallas import tpu_sc as plsc

N = 10000
E = 320000
CH = 128
NC = 2
NS = 16
EPT = E // NS
CHUNK = 128
CPS = 8
SPAN = CPS * CHUNK
NSPAN = 20
EPT_PAD = NSPAN * SPAN
EROW = NS * NSPAN * CPS
NROW = 640
ACC_ROWS = NROW * 16


def _vgather(a, idx):
  return a.at[idx].get(mode="promise_in_bounds")


def _sc_body(src_hbm, dst_hbm, ew_hbm, h_hbm, msg_hbm, deg_hbm,
             srcb, dstb, ewb, deg_v, rows0, rows1, msg_v, idrow_v,
             acc_sh, deg_sh, g0sem, g1sem, s0sem, s1sem):
  c = lax.axis_index("c")
  s = lax.axis_index("s")
  lanes = lax.iota(jnp.int32, 16)
  zi = lanes * 0
  zf = zi.astype(jnp.float32)
  row0 = s * (NSPAN * CPS)

  def _z_deg(r, carry):
    deg_v[r, :] = zf
    return carry
  lax.fori_loop(0, NROW, _z_deg, 0)

  def _z_rows(r, carry):
    for k in range(CH // 16):
      msg_v[r, pl.ds(k * 16, 16)] = zf
    return carry
  lax.fori_loop(0, CHUNK, _z_rows, 0)

  for v in range(5 * CHUNK // 16):
    idrow_v[v * 16 // CHUNK, pl.ds((v * 16) % CHUNK, 16)] = lanes + (v * 16)

  @pl.when(s == 0)
  def _():
    pltpu.sync_copy(deg_v, deg_sh)
  for k in range(ACC_ROWS // NS // CHUNK):
    pltpu.sync_copy(
        msg_v,
        acc_sh.at[pl.ds(s * (ACC_ROWS // NS) + k * CHUNK, CHUNK), :])

  up1 = jnp.minimum(lanes + 1, 15)

  def _deg_group(g, carry):
    d16 = dstb[g >> 3, pl.ds((g & 7) * 16, 16)]
    w16 = ewb[g >> 3, pl.ds((g & 7) * 16, 16)]
    sk, sv = plsc.sort_key_val(d16, w16)
    for k in (1, 2, 4, 8):
      idxk = jnp.maximum(lanes - k, 0)
      kg = _vgather(sk, idxk)
      vg = _vgather(sv, idxk)
      ok = (lanes >= k) & (kg == sk)
      sv = sv + jnp.where(ok, vg, 0.0)
    kn = _vgather(sk, up1)
    m = (sk != kn) | (lanes == 15)
    plsc.addupdate_scatter(deg_v, [sk >> 4, sk & 15], sv, mask=m)
    return carry

  def _deg_span(sp, carry):
    r0 = row0 + sp * CPS
    d1 = pltpu.async_copy(dst_hbm.at[pl.ds(r0, CPS), :], dstb, g0sem)
    d2 = pltpu.async_copy(ew_hbm.at[pl.ds(r0, CPS), :], ewb, g1sem)
    d1.wait()
    d2.wait()
    lax.fori_loop(0, SPAN // 16, _deg_group, 0)
    return carry
  lax.fori_loop(0, NSPAN, _deg_span, 0)
  plsc.subcore_barrier()

  for q in range(NROW // CHUNK):
    pltpu.sync_copy(deg_v.at[pl.ds(q * CHUNK, CHUNK), :],
                    deg_sh.at[idrow_v.at[q]], add=True)
  plsc.subcore_barrier()

  pltpu.sync_copy(deg_sh, deg_v)

  @pl.when((s == 0) & (c == 0))
  def _():
    pltpu.sync_copy(deg_v, deg_hbm)

  def _dinv_body(r, carry):
    dgt = deg_v[r, :] + 1.0
    xi = plsc.bitcast(dgt, jnp.int32)
    y = plsc.bitcast(jnp.int32(0x5F3759DF) - (xi >> 1), jnp.float32)
    hx = 0.5 * dgt
    for _ in range(3):
      y = y * (1.5 - hx * y * y)
    deg_v[r, :] = y
    return carry
  lax.fori_loop(0, NROW, _dinv_body, 0)

  def _norm_group(g, carry):
    s16 = srcb[g >> 3, pl.ds((g & 7) * 16, 16)]
    d16 = dstb[g >> 3, pl.ds((g & 7) * 16, 16)]
    w16 = ewb[g >> 3, pl.ds((g & 7) * 16, 16)]
    a = plsc.load_gather(deg_v, [s16 >> 4, s16 & 15])
    b = plsc.load_gather(deg_v, [d16 >> 4, d16 & 15])
    ewb[g >> 3, pl.ds((g & 7) * 16, 16)] = a * w16 * b
    srcb[g >> 3, pl.ds((g & 7) * 16, 16)] = s16 + c * N
    return carry

  rows = (rows0, rows1)
  gsem = (g0sem, g1sem)
  ssem = (s0sem,)

  def _msg_span(sp, carry):
    r0 = row0 + sp * CPS
    d1 = pltpu.async_copy(src_hbm.at[pl.ds(r0, CPS), :], srcb, g0sem)
    d2 = pltpu.async_copy(dst_hbm.at[pl.ds(r0, CPS), :], dstb, g1sem)
    d3 = pltpu.async_copy(ew_hbm.at[pl.ds(r0, CPS), :], ewb, s0sem)
    d1.wait()
    d2.wait()
    d3.wait()
    lax.fori_loop(0, SPAN // 16, _norm_group, 0)

    g_desc = [None, None]
    s_desc = [None]
    g_desc[0] = pltpu.async_copy(h_hbm.at[srcb.at[0]], rows[0], gsem[0])
    for q in range(CPS):
      b = q & 1
      nb = b ^ 1
      if q + 1 < CPS:
        g_desc[nb] = pltpu.async_copy(h_hbm.at[srcb.at[q + 1]], rows[nb],
                                      gsem[nb])
      g_desc[b].wait()
      if q >= 1:
        s_desc[0].wait()

      rb = rows[b]

      def _scale_body(g, carry2, _q=q, _rb=rb):
        nv = ewb[_q, pl.ds(g * 16, 16)]
        for j in range(16):
          w = _vgather(nv, zi + j)
          e = g * 16 + j
          for k in range(CH // 32):
            v32 = _rb[e, pl.ds(k * 32, 32)]
            lo, hi = plsc.unpack(v32, format=plsc.PackFormat.INTERLEAVED)
            msg_v[e, pl.ds(k * 32, 16)] = lo * w
            msg_v[e, pl.ds(k * 32 + 16, 16)] = hi * w
        return carry2
      lax.fori_loop(0, CHUNK // 16, _scale_body, 0)

      s_desc[0] = pltpu.async_copy(msg_v, acc_sh.at[dstb.at[q]], ssem[0],
                                   add=True)
    s_desc[0].wait()
    return carry
  lax.fori_loop(0, NSPAN, _msg_span, 0)

  plsc.subcore_barrier()

  for k in range(ACC_ROWS // NS // CHUNK):
    r0 = s * (ACC_ROWS // NS) + k * CHUNK
    pltpu.sync_copy(acc_sh.at[pl.ds(r0, CHUNK), :], msg_v)
    pltpu.sync_copy(msg_v, msg_hbm.at[c, pl.ds(r0, CHUNK), :])


_sc_message = functools.partial(
    pl.kernel,
    out_type=(jax.ShapeDtypeStruct((NC, ACC_ROWS, CH), jnp.float32),
              jax.ShapeDtypeStruct((NROW, 16), jnp.float32)),
    mesh=plsc.VectorSubcoreMesh(core_axis_name="c", subcore_axis_name="s"),
    compiler_params=pltpu.CompilerParams(
        needs_layout_passes=False, use_tc_tiling_on_sc=False),
    scratch_types=[
        pltpu.VMEM((CPS, CHUNK), jnp.int32),
        pltpu.VMEM((CPS, CHUNK), jnp.int32),
        pltpu.VMEM((CPS, CHUNK), jnp.float32),
        pltpu.VMEM((NROW, 16), jnp.float32),
        pltpu.VMEM((CHUNK, CH), jnp.bfloat16),
        pltpu.VMEM((CHUNK, CH), jnp.bfloat16),
        pltpu.VMEM((CHUNK, CH), jnp.float32),
        pltpu.VMEM((5, CHUNK), jnp.int32),
        pltpu.VMEM_SHARED((ACC_ROWS, CH), jnp.float32),
        pltpu.VMEM_SHARED((NROW, 16), jnp.float32),
        pltpu.SemaphoreType.DMA,
        pltpu.SemaphoreType.DMA,
        pltpu.SemaphoreType.DMA,
        pltpu.SemaphoreType.DMA,
    ],
)(_sc_body)


def _tc_h_body(x_ref, w_ref, wp_ref, o_ref, op_ref):
  xb = x_ref[...]
  o_ref[0] = jnp.dot(xb, w_ref[0], preferred_element_type=jnp.float32)
  op_ref[0] = jnp.dot(xb, wp_ref[0],
                      preferred_element_type=jnp.float32).astype(jnp.bfloat16)


_tc_h = pl.pallas_call(
    _tc_h_body,
    grid=(N // 1000, NC),
    in_specs=[
        pl.BlockSpec((1000, CH), lambda i, c: (i, 0)),
        pl.BlockSpec((1, CH, CH), lambda i, c: (c, 0, 0)),
        pl.BlockSpec((1, CH, CH), lambda i, c: (c, 0, 0)),
    ],
    out_specs=[
        pl.BlockSpec((1, 1000, CH), lambda i, c: (c, i, 0)),
        pl.BlockSpec((1, 1000, CH), lambda i, c: (c, i, 0)),
    ],
    out_shape=[
        jax.ShapeDtypeStruct((NC, N, CH), jnp.float32),
        jax.ShapeDtypeStruct((NC, N, CH), jnp.bfloat16),
    ],
)


def _tc_gates_body(m0, m1, h0, h1, deg, bz, bh, lz, lzb, lh, lhb, ow, ob,
                   o_ref):
  invd = 1.0 / (deg[...] + 1.0)
  cz = m0[0] + h0[...] * invd + bz[...]
  ch = m1[0] + h1[...] * invd + bh[...]
  z = jax.nn.sigmoid(
      jnp.dot(cz, lz[...], preferred_element_type=jnp.float32) + lzb[...])
  ht = jnp.tanh(
      jnp.dot(ch, lh[...], preferred_element_type=jnp.float32) + lhb[...])
  hn = (1.0 - z) * ht
  owv = ow[...]
  oweff = owv[:CH] + owv[CH:]
  y = jnp.dot(hn, oweff, preferred_element_type=jnp.float32) + ob[...]
  o_ref[...] = jax.nn.sigmoid(y)


_tc_gates = pl.pallas_call(
    _tc_gates_body,
    grid=(N // 1000,),
    in_specs=[
        pl.BlockSpec((1, 1000, CH), lambda i: (0, i, 0)),
        pl.BlockSpec((1, 1000, CH), lambda i: (1, i, 0)),
        pl.BlockSpec((1000, CH), lambda i: (i, 0)),
        pl.BlockSpec((1000, CH), lambda i: (i + N // 1000, 0)),
        pl.BlockSpec((1000, 1), lambda i: (i, 0)),
        pl.BlockSpec((CH,), lambda i: (0,)),
        pl.BlockSpec((CH,), lambda i: (0,)),
        pl.BlockSpec((CH, CH), lambda i: (0, 0)),
        pl.BlockSpec((CH,), lambda i: (0,)),
        pl.BlockSpec((CH, CH), lambda i: (0, 0)),
        pl.BlockSpec((CH,), lambda i: (0,)),
        pl.BlockSpec((2 * CH, 1), lambda i: (0, 0)),
        pl.BlockSpec((1,), lambda i: (0,)),
    ],
    out_specs=pl.BlockSpec((1000, 1), lambda i: (i, 0)),
    out_shape=jax.ShapeDtypeStruct((N, 1), jnp.float32),
)


def _pad_edges(a):
  return jnp.pad(a.reshape(NS, EPT),
                 ((0, 0), (0, EPT_PAD - EPT))).reshape(EROW, CHUNK)


_QPERM = np.empty((CH,), np.int32)
for _k in range(CH // 32):
  for _r in range(16):
    _QPERM[32 * _k + 2 * _r] = 32 * _k + _r
    _QPERM[32 * _k + 2 * _r + 1] = 32 * _k + 16 + _r


@jax.jit
def kernel(x, edge_index, edge_attr, Wz, bz, Wr, br, Wh, bh,
           Lzw, Lzb, Lrw, Lrb, Lhw, Lhb, Ow, Ob):
  src = _pad_edges(edge_index[0])
  dst = _pad_edges(edge_index[1])
  ew = _pad_edges(edge_attr)
  w3 = jnp.stack([Wz, Wh])
  w3p = w3[:, :, _QPERM]
  h, hp16 = _tc_h(x, w3, w3p)
  h2d = h.reshape(NC * N, CH)
  msg3, deg = _sc_message(src, dst, ew, hp16.reshape(NC * N, CH))
  deg_col = deg.reshape(-1)[:N].reshape(N, 1)
  return _tc_gates(msg3, msg3, h2d, h2d, deg_col,
                   bz, bh, Lzw, Lzb, Lhw, Lhb, Ow, Ob)


# speedup vs baseline: 1.3047x; 1.3047x over previous
import functools

import jax
import jax.numpy as jnp
from jax import lax
from jax.experimental import pallas as pl
from jax.experimental.pallas import tpu as pltpu
from jax.experimental.pallas import tpu_sc as plsc

N = 10000
E = 320000
CH = 128
NC = 2
NS = 16
EPT = E // NS
CHUNK = 128
CPS = 8
SPAN = CPS * CHUNK
NSPAN = 20
EPT_PAD = NSPAN * SPAN
EROW = NS * NSPAN * CPS
NROW = 640
ACC_ROWS = NROW * 16


def _vgather(a, idx):
  return a.at[idx].get(mode="promise_in_bounds")


def _sc_body(src_hbm, dst_hbm, ew_hbm, h_hbm, msg_hbm, deg_hbm,
             srcb, dstb, ewb, deg_v, rows0, rows1, idrow_v, acc_sh, deg_sh,
             *sems):
  c = lax.axis_index("c")
  s = lax.axis_index("s")
  lanes = lax.iota(jnp.int32, 16)
  zi = lanes * 0
  zf = zi.astype(jnp.float32)
  row0 = s * (NSPAN * CPS)

  def _z_deg(r, carry):
    deg_v[r, :] = zf
    return carry
  lax.fori_loop(0, NROW, _z_deg, 0)

  def _z_rows(r, carry):
    for k in range(CH // 16):
      rows0[r, pl.ds(k * 16, 16)] = zf
    return carry
  lax.fori_loop(0, CHUNK, _z_rows, 0)

  for v in range(5 * CHUNK // 16):
    idrow_v[v * 16 // CHUNK, pl.ds((v * 16) % CHUNK, 16)] = lanes + (v * 16)

  @pl.when(s == 0)
  def _():
    pltpu.sync_copy(deg_v, deg_sh)
  for k in range(ACC_ROWS // NS // CHUNK):
    pltpu.sync_copy(
        rows0,
        acc_sh.at[pl.ds(s * (ACC_ROWS // NS) + k * CHUNK, CHUNK), :])

  up1 = jnp.minimum(lanes + 1, 15)

  def _deg_group(g, carry):
    d16 = dstb[g >> 3, pl.ds((g & 7) * 16, 16)]
    w16 = ewb[g >> 3, pl.ds((g & 7) * 16, 16)]
    sk, sv = plsc.sort_key_val(d16, w16)
    for k in (1, 2, 4, 8):
      idxk = jnp.maximum(lanes - k, 0)
      kg = _vgather(sk, idxk)
      vg = _vgather(sv, idxk)
      ok = (lanes >= k) & (kg == sk)
      sv = sv + jnp.where(ok, vg, 0.0)
    kn = _vgather(sk, up1)
    m = (sk != kn) | (lanes == 15)
    plsc.addupdate_scatter(deg_v, [sk >> 4, sk & 15], sv, mask=m)
    return carry

  def _deg_span(sp, carry):
    r0 = row0 + sp * CPS
    d1 = pltpu.async_copy(dst_hbm.at[pl.ds(r0, CPS), :], dstb, sems[0])
    d2 = pltpu.async_copy(ew_hbm.at[pl.ds(r0, CPS), :], ewb, sems[1])
    d1.wait()
    d2.wait()
    lax.fori_loop(0, SPAN // 16, _deg_group, 0)
    return carry
  lax.fori_loop(0, NSPAN, _deg_span, 0)
  plsc.subcore_barrier()

  for q in range(NROW // CHUNK):
    pltpu.sync_copy(deg_v.at[pl.ds(q * CHUNK, CHUNK), :],
                    deg_sh.at[idrow_v.at[q]], add=True)
  plsc.subcore_barrier()

  pltpu.sync_copy(deg_sh, deg_v)

  @pl.when((s == 0) & (c == 0))
  def _():
    pltpu.sync_copy(deg_v, deg_hbm)

  def _dinv_body(r, carry):
    dgt = deg_v[r, :] + 1.0
    xi = plsc.bitcast(dgt, jnp.int32)
    y = plsc.bitcast(jnp.int32(0x5F3759DF) - (xi >> 1), jnp.float32)
    hx = 0.5 * dgt
    for _ in range(3):
      y = y * (1.5 - hx * y * y)
    deg_v[r, :] = y
    return carry
  lax.fori_loop(0, NROW, _dinv_body, 0)

  def _norm_group(g, carry):
    s16 = srcb[g >> 3, pl.ds((g & 7) * 16, 16)]
    d16 = dstb[g >> 3, pl.ds((g & 7) * 16, 16)]
    w16 = ewb[g >> 3, pl.ds((g & 7) * 16, 16)]
    a = plsc.load_gather(deg_v, [s16 >> 4, s16 & 15])
    b = plsc.load_gather(deg_v, [d16 >> 4, d16 & 15])
    ewb[g >> 3, pl.ds((g & 7) * 16, 16)] = a * w16 * b
    srcb[g >> 3, pl.ds((g & 7) * 16, 16)] = s16 + c * N
    return carry

  rows = (rows0, rows1)
  gsems = (sems[0:4], sems[4:8])
  ssem = (sems[8], sems[9])
  NSUB = 4
  SUB = CHUNK // NSUB

  def _msg_span(sp, carry):
    r0 = row0 + sp * CPS
    d1 = pltpu.async_copy(src_hbm.at[pl.ds(r0, CPS), :], srcb, sems[0])
    d2 = pltpu.async_copy(dst_hbm.at[pl.ds(r0, CPS), :], dstb, sems[1])
    d3 = pltpu.async_copy(ew_hbm.at[pl.ds(r0, CPS), :], ewb, sems[2])
    d1.wait()
    d2.wait()
    d3.wait()
    lax.fori_loop(0, SPAN // 16, _norm_group, 0)

    def _issue4(q, b):
      return [
          pltpu.async_copy(
              h_hbm.at[srcb.at[q, pl.ds(u * SUB, SUB)]],
              rows[b].at[pl.ds(u * SUB, SUB), :], gsems[b][u])
          for u in range(NSUB)
      ]

    g_desc = [None, None]
    s_desc = [None, None]
    g_desc[0] = _issue4(0, 0)
    for q in range(CPS):
      b = q & 1
      nb = b ^ 1
      if q + 1 < CPS:
        if q >= 1:
          s_desc[nb].wait()
        g_desc[nb] = _issue4(q + 1, nb)
      for d in g_desc[b]:
        d.wait()

      rb = rows[b]

      def _scale_body(g, carry2, _q=q, _rb=rb):
        nv = ewb[_q, pl.ds(g * 16, 16)]
        for j in range(16):
          w = _vgather(nv, zi + j)
          e = g * 16 + j
          for k in range(CH // 16):
            _rb[e, pl.ds(k * 16, 16)] = _rb[e, pl.ds(k * 16, 16)] * w
        return carry2
      lax.fori_loop(0, CHUNK // 16, _scale_body, 0)

      s_desc[b] = pltpu.async_copy(rb, acc_sh.at[dstb.at[q]], ssem[b],
                                   add=True)
    s_desc[0].wait()
    s_desc[1].wait()
    return carry
  lax.fori_loop(0, NSPAN, _msg_span, 0)

  plsc.subcore_barrier()

  for k in range(ACC_ROWS // NS // CHUNK):
    r0 = s * (ACC_ROWS // NS) + k * CHUNK
    pltpu.sync_copy(acc_sh.at[pl.ds(r0, CHUNK), :], rows0)
    pltpu.sync_copy(rows0, msg_hbm.at[c, pl.ds(r0, CHUNK), :])


_sc_message = functools.partial(
    pl.kernel,
    out_type=(jax.ShapeDtypeStruct((NC, ACC_ROWS, CH), jnp.float32),
              jax.ShapeDtypeStruct((NROW, 16), jnp.float32)),
    mesh=plsc.VectorSubcoreMesh(core_axis_name="c", subcore_axis_name="s"),
    compiler_params=pltpu.CompilerParams(
        needs_layout_passes=False, use_tc_tiling_on_sc=False),
    scratch_types=[
        pltpu.VMEM((CPS, CHUNK), jnp.int32),
        pltpu.VMEM((CPS, CHUNK), jnp.int32),
        pltpu.VMEM((CPS, CHUNK), jnp.float32),
        pltpu.VMEM((NROW, 16), jnp.float32),
        pltpu.VMEM((CHUNK, CH), jnp.float32),
        pltpu.VMEM((CHUNK, CH), jnp.float32),
        pltpu.VMEM((5, CHUNK), jnp.int32),
        pltpu.VMEM_SHARED((ACC_ROWS, CH), jnp.float32),
        pltpu.VMEM_SHARED((NROW, 16), jnp.float32),
    ] + [pltpu.SemaphoreType.DMA] * 10,
)(_sc_body)


def _tc_h_body(x_ref, w_ref, o_ref):
  o_ref[0] = jnp.dot(x_ref[...], w_ref[0],
                     preferred_element_type=jnp.float32)


_tc_h = pl.pallas_call(
    _tc_h_body,
    grid=(N // 1000, NC),
    in_specs=[
        pl.BlockSpec((1000, CH), lambda i, c: (i, 0)),
        pl.BlockSpec((1, CH, CH), lambda i, c: (c, 0, 0)),
    ],
    out_specs=pl.BlockSpec((1, 1000, CH), lambda i, c: (c, i, 0)),
    out_shape=jax.ShapeDtypeStruct((NC, N, CH), jnp.float32),
)


def _tc_gates_body(m0, m1, h0, h1, deg, bz, bh, lz, lzb, lh, lhb, ow, ob,
                   o_ref):
  invd = 1.0 / (deg[...] + 1.0)
  cz = m0[0] + h0[...] * invd + bz[...]
  ch = m1[0] + h1[...] * invd + bh[...]
  z = jax.nn.sigmoid(
      jnp.dot(cz, lz[...], preferred_element_type=jnp.float32) + lzb[...])
  ht = jnp.tanh(
      jnp.dot(ch, lh[...], preferred_element_type=jnp.float32) + lhb[...])
  hn = (1.0 - z) * ht
  owv = ow[...]
  oweff = owv[:CH] + owv[CH:]
  y = jnp.dot(hn, oweff, preferred_element_type=jnp.float32) + ob[...]
  o_ref[...] = jax.nn.sigmoid(y)


_tc_gates = pl.pallas_call(
    _tc_gates_body,
    grid=(N // 1000,),
    in_specs=[
        pl.BlockSpec((1, 1000, CH), lambda i: (0, i, 0)),
        pl.BlockSpec((1, 1000, CH), lambda i: (1, i, 0)),
        pl.BlockSpec((1000, CH), lambda i: (i, 0)),
        pl.BlockSpec((1000, CH), lambda i: (i + N // 1000, 0)),
        pl.BlockSpec((1000, 1), lambda i: (i, 0)),
        pl.BlockSpec((CH,), lambda i: (0,)),
        pl.BlockSpec((CH,), lambda i: (0,)),
        pl.BlockSpec((CH, CH), lambda i: (0, 0)),
        pl.BlockSpec((CH,), lambda i: (0,)),
        pl.BlockSpec((CH, CH), lambda i: (0, 0)),
        pl.BlockSpec((CH,), lambda i: (0,)),
        pl.BlockSpec((2 * CH, 1), lambda i: (0, 0)),
        pl.BlockSpec((1,), lambda i: (0,)),
    ],
    out_specs=pl.BlockSpec((1000, 1), lambda i: (i, 0)),
    out_shape=jax.ShapeDtypeStruct((N, 1), jnp.float32),
)


def _pad_edges(a):
  return jnp.pad(a.reshape(NS, EPT),
                 ((0, 0), (0, EPT_PAD - EPT))).reshape(EROW, CHUNK)


@jax.jit
def kernel(x, edge_index, edge_attr, Wz, bz, Wr, br, Wh, bh,
           Lzw, Lzb, Lrw, Lrb, Lhw, Lhb, Ow, Ob):
  src = _pad_edges(edge_index[0])
  dst = _pad_edges(edge_index[1])
  ew = _pad_edges(edge_attr)
  w3 = jnp.stack([Wz, Wh])
  h = _tc_h(x, w3)
  h2d = h.reshape(NC * N, CH)
  msg3, deg = _sc_message(src, dst, ew, h2d)
  deg_col = deg.reshape(-1)[:N].reshape(N, 1)
  return _tc_gates(msg3, msg3, h2d, h2d, deg_col,
                   bz, bh, Lzw, Lzb, Lhw, Lhb, Ow, Ob)


# speedup vs baseline: 1.3265x; 1.0167x over previous
import functools

import jax
import jax.numpy as jnp
from jax import lax
from jax.experimental import pallas as pl
from jax.experimental.pallas import tpu as pltpu
from jax.experimental.pallas import tpu_sc as plsc

N = 10000
E = 320000
CH = 128
NC = 2
NS = 16
EPT = E // NS
CHUNK = 128
CPS = 8
SPAN = CPS * CHUNK
NSPAN = 20
EPT_PAD = NSPAN * SPAN
EROW = NS * NSPAN * CPS
NROW = 640
ACC_ROWS = NROW * 16


def _vgather(a, idx):
  return a.at[idx].get(mode="promise_in_bounds")


def _sc_body(src_hbm, dst_hbm, ew_hbm, h_hbm, msg_hbm, deg_hbm,
             srcb, dstb, ewb, ewb2, deg_v, rows0, rows1, idrow_v, acc_sh,
             deg_sh, *sems):
  c = lax.axis_index("c")
  s = lax.axis_index("s")
  lanes = lax.iota(jnp.int32, 16)
  zi = lanes * 0
  zf = zi.astype(jnp.float32)
  row0 = s * (NSPAN * CPS)

  def _z_deg(r, carry):
    deg_v[r, :] = zf
    return carry
  lax.fori_loop(0, NROW, _z_deg, 0)

  def _z_rows(r, carry):
    for k in range(CH // 16):
      rows0[r, pl.ds(k * 16, 16)] = zf
    return carry
  lax.fori_loop(0, CHUNK, _z_rows, 0)

  for v in range(5 * CHUNK // 16):
    idrow_v[v * 16 // CHUNK, pl.ds((v * 16) % CHUNK, 16)] = lanes + (v * 16)

  @pl.when(s == 0)
  def _():
    pltpu.sync_copy(deg_v, deg_sh)
  for k in range(ACC_ROWS // NS // CHUNK):
    pltpu.sync_copy(
        rows0,
        acc_sh.at[pl.ds(s * (ACC_ROWS // NS) + k * CHUNK, CHUNK), :])

  up1 = jnp.minimum(lanes + 1, 15)

  def _make_deg_group(db, wb):
    def _deg_group(g, carry):
      d16 = db[g >> 3, pl.ds((g & 7) * 16, 16)]
      w16 = wb[g >> 3, pl.ds((g & 7) * 16, 16)]
      sk, sv = plsc.sort_key_val(d16, w16)
      for k in (1, 2, 4, 8):
        idxk = jnp.maximum(lanes - k, 0)
        kg = _vgather(sk, idxk)
        vg = _vgather(sv, idxk)
        ok = (lanes >= k) & (kg == sk)
        sv = sv + jnp.where(ok, vg, 0.0)
      kn = _vgather(sk, up1)
      m = (sk != kn) | (lanes == 15)
      plsc.addupdate_scatter(deg_v, [sk >> 4, sk & 15], sv, mask=m)
      return carry
    return _deg_group

  apairs = ((dstb, ewb), (srcb, ewb2))

  def _issue_span(sp, p):
    r0 = row0 + jnp.minimum(sp, NSPAN - 1) * CPS
    return (pltpu.async_copy(dst_hbm.at[pl.ds(r0, CPS), :], apairs[p][0],
                             sems[2 * p]),
            pltpu.async_copy(ew_hbm.at[pl.ds(r0, CPS), :], apairs[p][1],
                             sems[2 * p + 1]))

  adescs = [None, None]
  adescs[0] = _issue_span(0, 0)

  def _deg_span2(sp2, carry):
    for p in (0, 1):
      sp = sp2 * 2 + p
      nb = p ^ 1
      adescs[nb] = _issue_span(sp + 1, nb)
      for d in adescs[p]:
        d.wait()
      lax.fori_loop(0, SPAN // 16, _make_deg_group(*apairs[p]), 0)
    return carry
  lax.fori_loop(0, NSPAN // 2, _deg_span2, 0)
  pltpu.make_async_copy(dst_hbm.at[pl.ds(row0, CPS), :], apairs[0][0],
                        sems[0]).wait()
  pltpu.make_async_copy(ew_hbm.at[pl.ds(row0, CPS), :], apairs[0][1],
                        sems[1]).wait()
  plsc.subcore_barrier()

  for q in range(NROW // CHUNK):
    pltpu.sync_copy(deg_v.at[pl.ds(q * CHUNK, CHUNK), :],
                    deg_sh.at[idrow_v.at[q]], add=True)
  plsc.subcore_barrier()

  pltpu.sync_copy(deg_sh, deg_v)

  @pl.when((s == 0) & (c == 0))
  def _():
    pltpu.sync_copy(deg_v, deg_hbm)

  def _dinv_body(r, carry):
    dgt = deg_v[r, :] + 1.0
    xi = plsc.bitcast(dgt, jnp.int32)
    y = plsc.bitcast(jnp.int32(0x5F3759DF) - (xi >> 1), jnp.float32)
    hx = 0.5 * dgt
    for _ in range(3):
      y = y * (1.5 - hx * y * y)
    deg_v[r, :] = y
    return carry
  lax.fori_loop(0, NROW, _dinv_body, 0)

  def _norm_group(g, carry):
    s16 = srcb[g >> 3, pl.ds((g & 7) * 16, 16)]
    d16 = dstb[g >> 3, pl.ds((g & 7) * 16, 16)]
    w16 = ewb[g >> 3, pl.ds((g & 7) * 16, 16)]
    a = plsc.load_gather(deg_v, [s16 >> 4, s16 & 15])
    b = plsc.load_gather(deg_v, [d16 >> 4, d16 & 15])
    ewb[g >> 3, pl.ds((g & 7) * 16, 16)] = a * w16 * b
    srcb[g >> 3, pl.ds((g & 7) * 16, 16)] = s16 + c * N
    return carry

  rows = (rows0, rows1)
  gsems = (sems[0:4], sems[4:8])
  ssem = (sems[8], sems[9])
  NSUB = 4
  SUB = CHUNK // NSUB

  def _msg_span(sp, carry):
    r0 = row0 + sp * CPS
    d1 = pltpu.async_copy(src_hbm.at[pl.ds(r0, CPS), :], srcb, sems[0])
    d2 = pltpu.async_copy(dst_hbm.at[pl.ds(r0, CPS), :], dstb, sems[1])
    d3 = pltpu.async_copy(ew_hbm.at[pl.ds(r0, CPS), :], ewb, sems[2])
    d1.wait()
    d2.wait()
    d3.wait()
    lax.fori_loop(0, SPAN // 16, _norm_group, 0)

    def _issue4(q, b):
      return [
          pltpu.async_copy(
              h_hbm.at[srcb.at[q, pl.ds(u * SUB, SUB)]],
              rows[b].at[pl.ds(u * SUB, SUB), :], gsems[b][u])
          for u in range(NSUB)
      ]

    g_desc = [None, None]
    s_desc = [None, None]
    g_desc[0] = _issue4(0, 0)
    for q in range(CPS):
      b = q & 1
      nb = b ^ 1
      if q + 1 < CPS:
        if q >= 1:
          s_desc[nb].wait()
        g_desc[nb] = _issue4(q + 1, nb)
      for d in g_desc[b]:
        d.wait()

      rb = rows[b]

      def _scale_body(g, carry2, _q=q, _rb=rb):
        nv = ewb[_q, pl.ds(g * 16, 16)]
        for j in range(16):
          w = _vgather(nv, zi + j)
          e = g * 16 + j
          for k in range(CH // 16):
            _rb[e, pl.ds(k * 16, 16)] = _rb[e, pl.ds(k * 16, 16)] * w
        return carry2
      lax.fori_loop(0, CHUNK // 16, _scale_body, 0)

      s_desc[b] = pltpu.async_copy(rb, acc_sh.at[dstb.at[q]], ssem[b],
                                   add=True)
    s_desc[0].wait()
    s_desc[1].wait()
    return carry
  lax.fori_loop(0, NSPAN, _msg_span, 0)

  plsc.subcore_barrier()

  for k in range(ACC_ROWS // NS // CHUNK):
    r0 = s * (ACC_ROWS // NS) + k * CHUNK
    pltpu.sync_copy(acc_sh.at[pl.ds(r0, CHUNK), :], rows0)
    pltpu.sync_copy(rows0, msg_hbm.at[c, pl.ds(r0, CHUNK), :])


_sc_message = functools.partial(
    pl.kernel,
    out_type=(jax.ShapeDtypeStruct((NC, ACC_ROWS, CH), jnp.float32),
              jax.ShapeDtypeStruct((NROW, 16), jnp.float32)),
    mesh=plsc.VectorSubcoreMesh(core_axis_name="c", subcore_axis_name="s"),
    compiler_params=pltpu.CompilerParams(
        needs_layout_passes=False, use_tc_tiling_on_sc=False),
    scratch_types=[
        pltpu.VMEM((CPS, CHUNK), jnp.int32),
        pltpu.VMEM((CPS, CHUNK), jnp.int32),
        pltpu.VMEM((CPS, CHUNK), jnp.float32),
        pltpu.VMEM((CPS, CHUNK), jnp.float32),
        pltpu.VMEM((NROW, 16), jnp.float32),
        pltpu.VMEM((CHUNK, CH), jnp.float32),
        pltpu.VMEM((CHUNK, CH), jnp.float32),
        pltpu.VMEM((5, CHUNK), jnp.int32),
        pltpu.VMEM_SHARED((ACC_ROWS, CH), jnp.float32),
        pltpu.VMEM_SHARED((NROW, 16), jnp.float32),
    ] + [pltpu.SemaphoreType.DMA] * 10,
)(_sc_body)


def _tc_h_body(x_ref, w_ref, o_ref):
  o_ref[0] = jnp.dot(x_ref[...], w_ref[0],
                     preferred_element_type=jnp.float32)


_tc_h = pl.pallas_call(
    _tc_h_body,
    grid=(N // 1000, NC),
    in_specs=[
        pl.BlockSpec((1000, CH), lambda i, c: (i, 0)),
        pl.BlockSpec((1, CH, CH), lambda i, c: (c, 0, 0)),
    ],
    out_specs=pl.BlockSpec((1, 1000, CH), lambda i, c: (c, i, 0)),
    out_shape=jax.ShapeDtypeStruct((NC, N, CH), jnp.float32),
)


def _tc_gates_body(m0, m1, h0, h1, deg, bz, bh, lz, lzb, lh, lhb, ow, ob,
                   o_ref):
  invd = 1.0 / (deg[...] + 1.0)
  cz = m0[0] + h0[...] * invd + bz[...]
  ch = m1[0] + h1[...] * invd + bh[...]
  z = jax.nn.sigmoid(
      jnp.dot(cz, lz[...], preferred_element_type=jnp.float32) + lzb[...])
  ht = jnp.tanh(
      jnp.dot(ch, lh[...], preferred_element_type=jnp.float32) + lhb[...])
  hn = (1.0 - z) * ht
  owv = ow[...]
  oweff = owv[:CH] + owv[CH:]
  y = jnp.dot(hn, oweff, preferred_element_type=jnp.float32) + ob[...]
  o_ref[...] = jax.nn.sigmoid(y)


_tc_gates = pl.pallas_call(
    _tc_gates_body,
    grid=(N // 1000,),
    in_specs=[
        pl.BlockSpec((1, 1000, CH), lambda i: (0, i, 0)),
        pl.BlockSpec((1, 1000, CH), lambda i: (1, i, 0)),
        pl.BlockSpec((1000, CH), lambda i: (i, 0)),
        pl.BlockSpec((1000, CH), lambda i: (i + N // 1000, 0)),
        pl.BlockSpec((1000, 1), lambda i: (i, 0)),
        pl.BlockSpec((CH,), lambda i: (0,)),
        pl.BlockSpec((CH,), lambda i: (0,)),
        pl.BlockSpec((CH, CH), lambda i: (0, 0)),
        pl.BlockSpec((CH,), lambda i: (0,)),
        pl.BlockSpec((CH, CH), lambda i: (0, 0)),
        pl.BlockSpec((CH,), lambda i: (0,)),
        pl.BlockSpec((2 * CH, 1), lambda i: (0, 0)),
        pl.BlockSpec((1,), lambda i: (0,)),
    ],
    out_specs=pl.BlockSpec((1000, 1), lambda i: (i, 0)),
    out_shape=jax.ShapeDtypeStruct((N, 1), jnp.float32),
)


def _pad_edges(a):
  return jnp.pad(a.reshape(NS, EPT),
                 ((0, 0), (0, EPT_PAD - EPT))).reshape(EROW, CHUNK)


@jax.jit
def kernel(x, edge_index, edge_attr, Wz, bz, Wr, br, Wh, bh,
           Lzw, Lzb, Lrw, Lrb, Lhw, Lhb, Ow, Ob):
  src = _pad_edges(edge_index[0])
  dst = _pad_edges(edge_index[1])
  ew = _pad_edges(edge_attr)
  w3 = jnp.stack([Wz, Wh])
  h = _tc_h(x, w3)
  h2d = h.reshape(NC * N, CH)
  msg3, deg = _sc_message(src, dst, ew, h2d)
  deg_col = deg.reshape(-1)[:N].reshape(N, 1)
  return _tc_gates(msg3, msg3, h2d, h2d, deg_col,
                   bz, bh, Lzw, Lzb, Lhw, Lhb, Ow, Ob)
